# baseline (speedup 1.0000x reference)
import functools

import jax
import jax.numpy as jnp
from jax import lax
from jax.experimental import pallas as pl
from jax.experimental.pallas import tpu as pltpu
from jax.experimental.pallas import tpu_sc as plsc

NC, NS = 2, 16
NW = NC * NS
LANES = 16
CHUNK = 80


def _proj_body(h_ref, ws_ref, wt_ref, os_ref, ot_ref):
    hb = h_ref[...]
    os_ref[...] = jnp.dot(hb, ws_ref[...], preferred_element_type=jnp.float32)
    ot_ref[...] = jnp.dot(hb, wt_ref[...], preferred_element_type=jnp.float32)


def _project(h, w1s, w1t, blk=1000):
    n, d = h.shape
    grid = (n // blk,)
    return pl.pallas_call(
        _proj_body,
        grid=grid,
        in_specs=[
            pl.BlockSpec((blk, d), lambda i: (i, 0)),
            pl.BlockSpec((d, d), lambda i: (0, 0)),
            pl.BlockSpec((d, d), lambda i: (0, 0)),
        ],
        out_specs=[
            pl.BlockSpec((blk, d), lambda i: (i, 0)),
            pl.BlockSpec((blk, d), lambda i: (i, 0)),
        ],
        out_shape=[
            jax.ShapeDtypeStruct((n, d), jnp.float32),
            jax.ShapeDtypeStruct((n, d), jnp.float32),
        ],
    )(h, w1s, w1t)


NBUF = 5


def _make_gather(E, N, D, e_base, ck=80, nb=5):
    epw = E // NW
    steps = epw // ck
    groups = steps // nb
    mesh = plsc.VectorSubcoreMesh(core_axis_name="c", subcore_axis_name="s")

    @functools.partial(
        pl.kernel,
        out_type=jax.ShapeDtypeStruct((E, D), jnp.float32),
        mesh=mesh,
        scratch_types=(
            [pltpu.VMEM((ck,), jnp.int32) for _ in range(nb)]
            + [pltpu.VMEM((ck,), jnp.int32) for _ in range(nb)]
            + [pltpu.VMEM((ck, D), jnp.float32) for _ in range(nb)]
            + [pltpu.SemaphoreType.DMA for _ in range(2 * nb)]
        ),
    )
    def gather_k(hs_hbm, ht_hbm, row_hbm, col_hbm, out_hbm, *scr):
        idx_r = scr[:nb]
        idx_c = scr[nb:2 * nb]
        buf = scr[2 * nb:3 * nb]
        sem = scr[3 * nb:4 * nb]
        sem2 = scr[4 * nb:5 * nb]
        wid = lax.axis_index("s") * NC + lax.axis_index("c")
        base = wid * epw

        def body(j, carry):
            offs = [pl.multiple_of(base + (j * nb + b) * ck, 8)
                    for b in range(nb)]
            ia, ib = [], []
            for b in range(nb):
                roff = pl.multiple_of(e_base + offs[b], 8)
                ia.append(pltpu.async_copy(
                    row_hbm.at[pl.ds(roff, ck)], idx_r[b], sem[b]))
                ib.append(pltpu.async_copy(
                    col_hbm.at[pl.ds(roff, ck)], idx_c[b], sem2[b]))
            ga = []
            for b in range(nb):
                ia[b].wait()
                ga.append(pltpu.async_copy(
                    hs_hbm.at[idx_r[b]], buf[b], sem[b]))
            gb = []
            for b in range(nb):
                ga[b].wait()
                ib[b].wait()
                gb.append(pltpu.async_copy(
                    ht_hbm.at[idx_c[b]], buf[b], sem[b], add=True))
            wo = []
            for b in range(nb):
                gb[b].wait()
                wo.append(pltpu.async_copy(
                    buf[b], out_hbm.at[pl.ds(offs[b], ck)], sem[b]))
            for b in range(nb):
                wo[b].wait()
            return carry

        lax.fori_loop(0, groups, body, 0)

    return gather_k


def _edge_mlp_body(alias_ref, pre_ref, ea_ref, w1a_ref, b1_ref, g_ref,
                   bb_ref, w2_ref, b2_ref, out_ref):
    del alias_ref
    x = pre_ref[...]
    x = x + lax.dot_general(ea_ref[...], w1a_ref[...],
                            (((0,), (0,)), ((), ())),
                            preferred_element_type=jnp.float32)
    x = x + b1_ref[...]
    mu = jnp.mean(x, axis=-1, keepdims=True)
    xc = x - mu
    var = jnp.mean(xc * xc, axis=-1, keepdims=True)
    y = xc / jnp.sqrt(var + 1e-5) * g_ref[...] + bb_ref[...]
    m = y * jax.nn.sigmoid(y)
    z = jnp.dot(m, w2_ref[...], preferred_element_type=jnp.float32)
    z = z + b2_ref[...]
    out_ref[...] = z * jax.nn.sigmoid(z)


def _edge_mlp_slice(mij_buf, pre_s, ea_t, w1a, b1, g, bb, w2, b2,
                    e_total, sb, blk=2560):
    es, d = pre_s.shape
    de = ea_t.shape[0]
    grid = (es // blk,)
    first = mij_buf is None
    if first:
        mij_buf = pre_s
    return pl.pallas_call(
        _edge_mlp_body,
        grid=grid,
        in_specs=[
            pl.BlockSpec(memory_space=pl.ANY),
            pl.BlockSpec((blk, d), lambda i: (i, 0)),
            pl.BlockSpec((de, blk), lambda i: (0, sb + i)),
            pl.BlockSpec((de, d), lambda i: (0, 0)),
            pl.BlockSpec((d,), lambda i: (0,)),
            pl.BlockSpec((d,), lambda i: (0,)),
            pl.BlockSpec((d,), lambda i: (0,)),
            pl.BlockSpec((d, d), lambda i: (0, 0)),
            pl.BlockSpec((d,), lambda i: (0,)),
        ],
        out_specs=pl.BlockSpec((blk, d), lambda i: (sb + i, 0)),
        out_shape=jax.ShapeDtypeStruct((e_total, d), jnp.float32),
        input_output_aliases={} if first else {0: 0},
        compiler_params=pltpu.CompilerParams(
            dimension_semantics=("arbitrary",)),
    )(mij_buf, pre_s, ea_t, w1a, b1, g, bb, w2, b2)


def _make_scatter(E, N, D):
    epw = E // NW
    ck = 40
    steps = epw // ck
    npt = (N // NS) // 8 * 8
    tail = N - NS * npt
    mesh = plsc.VectorSubcoreMesh(core_axis_name="c", subcore_axis_name="s")

    @functools.partial(
        pl.kernel,
        out_type=jax.ShapeDtypeStruct((NC, N, D), jnp.float32),
        mesh=mesh,
        compiler_params=pltpu.CompilerParams(use_tc_tiling_on_sc=True),
        scratch_types=(
            [pltpu.VMEM((ck,), jnp.int32) for _ in range(NBUF)]
            + [pltpu.VMEM((ck, D), jnp.float32) for _ in range(NBUF)]
            + [pltpu.VMEM_SHARED((N, D), jnp.float32)]
            + [pltpu.SemaphoreType.DMA for _ in range(2 * NBUF)]
        ),
    )
    def scatter_k(mij_hbm, row_hbm, zeros_hbm, out_hbm, *scr):
        idx = scr[:NBUF]
        buf = scr[NBUF:2 * NBUF]
        shared = scr[2 * NBUF]
        semi = scr[2 * NBUF + 1:3 * NBUF + 1]
        semm = scr[3 * NBUF + 1:4 * NBUF + 1]
        cid = lax.axis_index("c")
        sid = lax.axis_index("s")
        wid = sid * NC + cid
        noff = pl.multiple_of(sid * npt, 8)
        pltpu.sync_copy(zeros_hbm.at[pl.ds(noff, npt)],
                        shared.at[pl.ds(noff, npt)])

        @pl.when(sid == NS - 1)
        def _init_tail():
            pltpu.sync_copy(zeros_hbm.at[pl.ds(NS * npt, tail)],
                            shared.at[pl.ds(NS * npt, tail)])

        plsc.subcore_barrier()
        base = wid * epw
        groups = steps // NBUF

        def body(j, carry):
            offs = [pl.multiple_of(base + (j * NBUF + b) * ck, 8)
                    for b in range(NBUF)]
            ii, mm = [], []
            for b in range(NBUF):
                ii.append(pltpu.async_copy(
                    row_hbm.at[pl.ds(offs[b], ck)], idx[b], semi[b]))
                mm.append(pltpu.async_copy(
                    mij_hbm.at[pl.ds(offs[b], ck)], buf[b], semm[b]))
            sc = []
            for b in range(NBUF):
                ii[b].wait()
                mm[b].wait()
                sc.append(pltpu.async_copy(
                    buf[b], shared.at[idx[b]], semi[b], add=True))
            for b in range(NBUF):
                sc[b].wait()
            return carry

        lax.fori_loop(0, groups, body, 0)
        plsc.subcore_barrier()
        pltpu.sync_copy(shared.at[pl.ds(noff, npt)],
                        out_hbm.at[cid, pl.ds(noff, npt)])

        @pl.when(sid == NS - 1)
        def _drain_tail():
            pltpu.sync_copy(shared.at[pl.ds(NS * npt, tail)],
                            out_hbm.at[cid, pl.ds(NS * npt, tail)])

    return scatter_k


def _node_mlp_body(h_ref, a0_ref, a1_ref, wnh_ref, wna_ref, b1_ref,
                   g_ref, bb_ref, w2_ref, b2_ref, out_ref):
    hb = h_ref[...]
    agg = (a0_ref[...] + a1_ref[...]) * 0.01
    x = jnp.dot(hb, wnh_ref[...], preferred_element_type=jnp.float32)
    x = x + jnp.dot(agg, wna_ref[...], preferred_element_type=jnp.float32)
    x = x + b1_ref[...]
    mu = jnp.mean(x, axis=-1, keepdims=True)
    xc = x - mu
    var = jnp.mean(xc * xc, axis=-1, keepdims=True)
    y = xc / jnp.sqrt(var + 1e-5) * g_ref[...] + bb_ref[...]
    nh = y * jax.nn.sigmoid(y)
    z = jnp.dot(nh, w2_ref[...], preferred_element_type=jnp.float32)
    out_ref[...] = hb + z + b2_ref[...]


def _node_mlp(h, a0, a1, wnh, wna, b1, g, bb, w2, b2, blk=1000):
    n, d = h.shape
    grid = (n // blk,)
    full = lambda i: (0, 0)
    vec = lambda i: (0,)
    rows = lambda i: (i, 0)
    return pl.pallas_call(
        _node_mlp_body,
        grid=grid,
        in_specs=[
            pl.BlockSpec((blk, d), rows),
            pl.BlockSpec((blk, d), rows),
            pl.BlockSpec((blk, d), rows),
            pl.BlockSpec((d, d), full),
            pl.BlockSpec((d, d), full),
            pl.BlockSpec((d,), vec),
            pl.BlockSpec((d,), vec),
            pl.BlockSpec((d,), vec),
            pl.BlockSpec((d, d), full),
            pl.BlockSpec((d,), vec),
        ],
        out_specs=pl.BlockSpec((blk, d), rows),
        out_shape=jax.ShapeDtypeStruct((n, d), jnp.float32),
    )(h, a0, a1, wnh, wna, b1, g, bb, w2, b2)


def kernel(h, edge_index, edge_attr, W_e1, b_e1, g_e, bb_e, W_e2, b_e2,
           W_n1, b_n1, g_n, bb_n, W_n2, b_n2):
    n, d = h.shape
    e = edge_index.shape[1]
    row = edge_index[0]
    col = edge_index[1]

    w1s = W_e1[:d]
    w1t = W_e1[d:2 * d]
    w1a = W_e1[2 * d:]
    wnh = W_n1[:d]
    wna = W_n1[d:]

    ea_t = edge_attr.T
    hs, ht = _project(h, w1s, w1t)

    n_slices = 10
    es = e // n_slices
    pres = [_make_gather(es, n, d, s * es, ck=40, nb=5)(hs, ht, row, col)
            for s in range(n_slices)]
    blk = 3200
    mij = None
    for s in range(n_slices):
        mij = _edge_mlp_slice(mij, pres[s], ea_t, w1a, b_e1, g_e,
                              bb_e, W_e2, b_e2, e, s * es // blk, blk)
    parts = _make_scatter(e, n, d)(mij, row, jnp.zeros((n, d), jnp.float32))
    h_out = _node_mlp(h, parts[0], parts[1], wnh, wna, b_n1, g_n, bb_n,
                      W_n2, b_n2)
    return (h_out, mij)


# speedup vs baseline: 1.0406x; 1.0406x over previous
import functools

import jax
import jax.numpy as jnp
from jax import lax
from jax.experimental import pallas as pl
from jax.experimental.pallas import tpu as pltpu
from jax.experimental.pallas import tpu_sc as plsc

NC, NS = 2, 16
NW = NC * NS
LANES = 16
CHUNK = 80


def _proj_body(h_ref, ws_ref, wt_ref, os_ref, ot_ref):
    hb = h_ref[...]
    os_ref[...] = jnp.dot(hb, ws_ref[...], preferred_element_type=jnp.float32)
    ot_ref[...] = jnp.dot(hb, wt_ref[...], preferred_element_type=jnp.float32)


def _project(h, w1s, w1t, blk=1000):
    n, d = h.shape
    grid = (n // blk,)
    return pl.pallas_call(
        _proj_body,
        grid=grid,
        in_specs=[
            pl.BlockSpec((blk, d), lambda i: (i, 0)),
            pl.BlockSpec((d, d), lambda i: (0, 0)),
            pl.BlockSpec((d, d), lambda i: (0, 0)),
        ],
        out_specs=[
            pl.BlockSpec((blk, d), lambda i: (i, 0)),
            pl.BlockSpec((blk, d), lambda i: (i, 0)),
        ],
        out_shape=[
            jax.ShapeDtypeStruct((n, d), jnp.float32),
            jax.ShapeDtypeStruct((n, d), jnp.float32),
        ],
    )(h, w1s, w1t)


NBUF = 5


def _make_gather(E, N, D, e_base, ck=80, nb=5):
    epw = E // NW
    steps = epw // ck
    groups = steps // nb
    mesh = plsc.VectorSubcoreMesh(core_axis_name="c", subcore_axis_name="s")

    @functools.partial(
        pl.kernel,
        out_type=jax.ShapeDtypeStruct((E, D), jnp.float32),
        mesh=mesh,
        scratch_types=(
            [pltpu.VMEM((ck,), jnp.int32) for _ in range(nb)]
            + [pltpu.VMEM((ck,), jnp.int32) for _ in range(nb)]
            + [pltpu.VMEM((ck, D), jnp.float32) for _ in range(nb)]
            + [pltpu.SemaphoreType.DMA for _ in range(2 * nb)]
        ),
    )
    def gather_k(hs_hbm, ht_hbm, row_hbm, col_hbm, out_hbm, *scr):
        idx_r = scr[:nb]
        idx_c = scr[nb:2 * nb]
        buf = scr[2 * nb:3 * nb]
        sem = scr[3 * nb:4 * nb]
        sem2 = scr[4 * nb:5 * nb]
        wid = lax.axis_index("s") * NC + lax.axis_index("c")
        base = wid * epw

        def body(j, carry):
            offs = [pl.multiple_of(base + (j * nb + b) * ck, 8)
                    for b in range(nb)]
            ia, ib = [], []
            for b in range(nb):
                roff = pl.multiple_of(e_base + offs[b], 8)
                ia.append(pltpu.async_copy(
                    row_hbm.at[pl.ds(roff, ck)], idx_r[b], sem[b]))
                ib.append(pltpu.async_copy(
                    col_hbm.at[pl.ds(roff, ck)], idx_c[b], sem2[b]))
            ga = []
            for b in range(nb):
                ia[b].wait()
                ga.append(pltpu.async_copy(
                    hs_hbm.at[idx_r[b]], buf[b], sem[b]))
            gb = []
            for b in range(nb):
                ga[b].wait()
                ib[b].wait()
                gb.append(pltpu.async_copy(
                    ht_hbm.at[idx_c[b]], buf[b], sem[b], add=True))
            wo = []
            for b in range(nb):
                gb[b].wait()
                wo.append(pltpu.async_copy(
                    buf[b], out_hbm.at[pl.ds(offs[b], ck)], sem[b]))
            for b in range(nb):
                wo[b].wait()
            return carry

        lax.fori_loop(0, groups, body, 0)

    return gather_k


def _edge_mlp_body(alias_ref, pre_ref, ea_ref, w1a_ref, b1_ref, g_ref,
                   bb_ref, w2_ref, b2_ref, out_ref):
    del alias_ref
    x = pre_ref[...]
    x = x + lax.dot_general(ea_ref[...], w1a_ref[...],
                            (((0,), (0,)), ((), ())),
                            preferred_element_type=jnp.float32)
    x = x + b1_ref[...]
    mu = jnp.mean(x, axis=-1, keepdims=True)
    xc = x - mu
    var = jnp.mean(xc * xc, axis=-1, keepdims=True)
    y = xc / jnp.sqrt(var + 1e-5) * g_ref[...] + bb_ref[...]
    m = y * jax.nn.sigmoid(y)
    z = jnp.dot(m, w2_ref[...], preferred_element_type=jnp.float32)
    z = z + b2_ref[...]
    out_ref[...] = z * jax.nn.sigmoid(z)


def _edge_mlp_slice(mij_buf, pre_s, ea_t, w1a, b1, g, bb, w2, b2,
                    e_total, sb, blk=3200):
    es, d = pre_s.shape
    de = ea_t.shape[0]
    grid = (es // blk,)
    first = mij_buf is None
    if first:
        mij_buf = pre_s
    return pl.pallas_call(
        _edge_mlp_body,
        grid=grid,
        in_specs=[
            pl.BlockSpec(memory_space=pl.ANY),
            pl.BlockSpec((blk, d), lambda i: (i, 0)),
            pl.BlockSpec((de, blk), lambda i: (0, sb + i)),
            pl.BlockSpec((de, d), lambda i: (0, 0)),
            pl.BlockSpec((d,), lambda i: (0,)),
            pl.BlockSpec((d,), lambda i: (0,)),
            pl.BlockSpec((d,), lambda i: (0,)),
            pl.BlockSpec((d, d), lambda i: (0, 0)),
            pl.BlockSpec((d,), lambda i: (0,)),
        ],
        out_specs=pl.BlockSpec((blk, d), lambda i: (sb + i, 0)),
        out_shape=jax.ShapeDtypeStruct((e_total, d), jnp.float32),
        input_output_aliases={} if first else {0: 0},
        compiler_params=pltpu.CompilerParams(
            dimension_semantics=("arbitrary",)),
    )(mij_buf, pre_s, ea_t, w1a, b1, g, bb, w2, b2)


def _make_scatter(E, N, D):
    epw = E // NW
    ck = 40
    steps = epw // ck
    npt = (N // NS) // 8 * 8
    tail = N - NS * npt
    mesh = plsc.VectorSubcoreMesh(core_axis_name="c", subcore_axis_name="s")

    @functools.partial(
        pl.kernel,
        out_type=jax.ShapeDtypeStruct((NC, N, D), jnp.float32),
        mesh=mesh,
        compiler_params=pltpu.CompilerParams(use_tc_tiling_on_sc=True),
        scratch_types=(
            [pltpu.VMEM((ck,), jnp.int32) for _ in range(NBUF)]
            + [pltpu.VMEM((ck, D), jnp.float32) for _ in range(NBUF)]
            + [pltpu.VMEM_SHARED((N, D), jnp.float32)]
            + [pltpu.SemaphoreType.DMA for _ in range(2 * NBUF)]
        ),
    )
    def scatter_k(mij_hbm, row_hbm, zeros_hbm, out_hbm, *scr):
        idx = scr[:NBUF]
        buf = scr[NBUF:2 * NBUF]
        shared = scr[2 * NBUF]
        semi = scr[2 * NBUF + 1:3 * NBUF + 1]
        semm = scr[3 * NBUF + 1:4 * NBUF + 1]
        cid = lax.axis_index("c")
        sid = lax.axis_index("s")
        wid = sid * NC + cid
        noff = pl.multiple_of(sid * npt, 8)
        pltpu.sync_copy(zeros_hbm.at[pl.ds(noff, npt)],
                        shared.at[pl.ds(noff, npt)])

        @pl.when(sid == NS - 1)
        def _init_tail():
            pltpu.sync_copy(zeros_hbm.at[pl.ds(NS * npt, tail)],
                            shared.at[pl.ds(NS * npt, tail)])

        plsc.subcore_barrier()
        base = wid * epw
        groups = steps // NBUF

        def body(j, carry):
            offs = [pl.multiple_of(base + (j * NBUF + b) * ck, 8)
                    for b in range(NBUF)]
            ii, mm = [], []
            for b in range(NBUF):
                ii.append(pltpu.async_copy(
                    row_hbm.at[pl.ds(offs[b], ck)], idx[b], semi[b]))
                mm.append(pltpu.async_copy(
                    mij_hbm.at[pl.ds(offs[b], ck)], buf[b], semm[b]))
            sc = []
            for b in range(NBUF):
                ii[b].wait()
                mm[b].wait()
                sc.append(pltpu.async_copy(
                    buf[b], shared.at[idx[b]], semi[b], add=True))
            for b in range(NBUF):
                sc[b].wait()
            return carry

        lax.fori_loop(0, groups, body, 0)
        plsc.subcore_barrier()
        pltpu.sync_copy(shared.at[pl.ds(noff, npt)],
                        out_hbm.at[cid, pl.ds(noff, npt)])

        @pl.when(sid == NS - 1)
        def _drain_tail():
            pltpu.sync_copy(shared.at[pl.ds(NS * npt, tail)],
                            out_hbm.at[cid, pl.ds(NS * npt, tail)])

    return scatter_k


def _node_mlp_body(h_ref, a0_ref, a1_ref, wnh_ref, wna_ref, b1_ref,
                   g_ref, bb_ref, w2_ref, b2_ref, out_ref):
    hb = h_ref[...]
    agg = (a0_ref[...] + a1_ref[...]) * 0.01
    x = jnp.dot(hb, wnh_ref[...], preferred_element_type=jnp.float32)
    x = x + jnp.dot(agg, wna_ref[...], preferred_element_type=jnp.float32)
    x = x + b1_ref[...]
    mu = jnp.mean(x, axis=-1, keepdims=True)
    xc = x - mu
    var = jnp.mean(xc * xc, axis=-1, keepdims=True)
    y = xc / jnp.sqrt(var + 1e-5) * g_ref[...] + bb_ref[...]
    nh = y * jax.nn.sigmoid(y)
    z = jnp.dot(nh, w2_ref[...], preferred_element_type=jnp.float32)
    out_ref[...] = hb + z + b2_ref[...]


def _node_mlp(h, a0, a1, wnh, wna, b1, g, bb, w2, b2, blk=1000):
    n, d = h.shape
    grid = (n // blk,)
    full = lambda i: (0, 0)
    vec = lambda i: (0,)
    rows = lambda i: (i, 0)
    return pl.pallas_call(
        _node_mlp_body,
        grid=grid,
        in_specs=[
            pl.BlockSpec((blk, d), rows),
            pl.BlockSpec((blk, d), rows),
            pl.BlockSpec((blk, d), rows),
            pl.BlockSpec((d, d), full),
            pl.BlockSpec((d, d), full),
            pl.BlockSpec((d,), vec),
            pl.BlockSpec((d,), vec),
            pl.BlockSpec((d,), vec),
            pl.BlockSpec((d, d), full),
            pl.BlockSpec((d,), vec),
        ],
        out_specs=pl.BlockSpec((blk, d), rows),
        out_shape=jax.ShapeDtypeStruct((n, d), jnp.float32),
    )(h, a0, a1, wnh, wna, b1, g, bb, w2, b2)


def kernel(h, edge_index, edge_attr, W_e1, b_e1, g_e, bb_e, W_e2, b_e2,
           W_n1, b_n1, g_n, bb_n, W_n2, b_n2):
    n, d = h.shape
    e = edge_index.shape[1]
    row = edge_index[0]
    col = edge_index[1]

    w1s = W_e1[:d]
    w1t = W_e1[d:2 * d]
    w1a = W_e1[2 * d:]
    wnh = W_n1[:d]
    wna = W_n1[d:]

    ea_t = edge_attr.T
    hs, ht = _project(h, w1s, w1t)

    n_slices = 5
    es = e // n_slices
    pres = [_make_gather(es, n, d, s * es)(hs, ht, row, col)
            for s in range(n_slices)]
    blk = 3200
    mij = None
    for s in range(n_slices):
        mij = _edge_mlp_slice(mij, pres[s], ea_t, w1a, b_e1, g_e,
                              bb_e, W_e2, b_e2, e, s * es // blk, blk)
    parts = _make_scatter(e, n, d)(mij, row, jnp.zeros((n, d), jnp.float32))
    h_out = _node_mlp(h, parts[0], parts[1], wnh, wna, b_n1, g_n, bb_n,
                      W_n2, b_n2)
    return (h_out, mij)
